# Initial kernel scaffold; baseline (speedup 1.0000x reference)
import jax
import jax.numpy as jnp
from jax.experimental import pallas as pl


def kernel(concept_ids, embeddings, gamma_phases, t):
    raise NotImplementedError("write your pallas kernel here")


# speedup vs baseline: 14.7072x; 14.7072x over previous
import functools
import math

import jax
import jax.numpy as jnp
from jax import lax
from jax.experimental import pallas as pl
from jax.experimental.pallas import tpu as pltpu
from jax.experimental.pallas import tpu_sc as plsc

_THETA_FREQ = 6.0
_GAMMA_FREQ = 40.0
_D = 64

_NC = 2
_NS = 16
_NW = _NC * _NS


def _mod_table_body(t_ref, emb_ref, gp_ref, out_ref):
    t = t_ref[0, 0]
    theta_mod = 0.5 + 0.5 * jnp.cos(2.0 * math.pi * _THETA_FREQ * t)
    gamma_t = 2.0 * math.pi * _GAMMA_FREQ * t
    scale = theta_mod * (0.5 + 0.5 * jnp.cos(gamma_t - gp_ref[:, :]))
    out_ref[:, :] = emb_ref[:, :] * scale


def _modulated_table(embeddings, gamma_phases, t):
    n = embeddings.shape[0]
    t_arr = jnp.reshape(t, (1, 1)).astype(jnp.float32)
    gp2d = gamma_phases.reshape(n, 1)
    return pl.pallas_call(
        _mod_table_body,
        out_shape=jax.ShapeDtypeStruct((n, _D), jnp.float32),
        in_specs=[
            pl.BlockSpec(memory_space=pltpu.SMEM),
            pl.BlockSpec(memory_space=pltpu.VMEM),
            pl.BlockSpec(memory_space=pltpu.VMEM),
        ],
    )(t_arr, embeddings, gp2d)


def _sc_gather(flat_ids, table, n_rows, chunk):
    n_chunks = n_rows // (_NW * chunk)
    per_w = n_rows // _NW
    mesh = plsc.VectorSubcoreMesh(core_axis_name="c", subcore_axis_name="s")

    @functools.partial(
        pl.kernel,
        out_type=jax.ShapeDtypeStruct((n_rows, _D), jnp.float32),
        mesh=mesh,
        scratch_types=[
            pltpu.VMEM((chunk,), jnp.int32),
            pltpu.VMEM((chunk, _D), jnp.float32),
            pltpu.SemaphoreType.DMA,
        ],
        compiler_params=pltpu.CompilerParams(use_tc_tiling_on_sc=False),
    )
    def k(idx_hbm, table_hbm, out_hbm, idx_v, rows_v, sem):
        wid = lax.axis_index("s") * _NC + lax.axis_index("c")
        w_base = wid * per_w

        def body(i, carry):
            base = w_base + i * chunk
            pltpu.sync_copy(idx_hbm.at[pl.ds(base, chunk)], idx_v)
            pltpu.async_copy(table_hbm.at[idx_v], rows_v, sem).wait()
            pltpu.sync_copy(rows_v, out_hbm.at[pl.ds(base, chunk)])
            return carry

        lax.fori_loop(0, n_chunks, body, 0)

    return k(flat_ids, table)


def kernel(concept_ids, embeddings, gamma_phases, t):
    table = _modulated_table(embeddings, gamma_phases, t)
    flat = concept_ids.reshape(-1).astype(jnp.int32)
    n_rows = flat.shape[0]
    out = _sc_gather(flat, table, n_rows, chunk=512)
    return out.reshape(concept_ids.shape + (_D,))
